# baseline (speedup 1.0000x reference)
import functools

import jax
import jax.numpy as jnp
from jax.experimental import pallas as pl
from jax.experimental.pallas import tpu as pltpu


def _copy_body(w_ref, o_ref, sem):
    copy = pltpu.make_async_copy(w_ref, o_ref, sem)
    copy.start()
    copy.wait()


@functools.cache
def _tc_copy(seq_len: int, emb_dim: int):
    return pl.pallas_call(
        _copy_body,
        out_shape=jax.ShapeDtypeStruct((seq_len, emb_dim), jnp.float32),
        out_specs=pl.BlockSpec(memory_space=pl.ANY),
        scratch_shapes=[pltpu.SemaphoreType.DMA],
        compiler_params=pltpu.CompilerParams(
            disable_bounds_checks=True,
            disable_semaphore_checks=True,
            skip_device_barrier=True,
        ),
    )


def kernel(x, wpe):
    seq_len = x.shape[1]
    emb_dim = wpe.shape[1]
    out = _tc_copy(seq_len, emb_dim)(wpe[:seq_len])
    return jnp.reshape(out, (1, seq_len, emb_dim))


# speedup vs baseline: 1.0053x; 1.0053x over previous
import functools

import jax
import jax.numpy as jnp
from jax.experimental import pallas as pl
from jax.experimental.pallas import tpu as pltpu


def _copy_body(w_ref, o_ref, sem):
    copy = pltpu.make_async_copy(w_ref, o_ref.at[0], sem)
    copy.start()
    copy.wait()


@functools.cache
def _tc_copy(seq_len: int, emb_dim: int):
    return pl.pallas_call(
        _copy_body,
        out_shape=jax.ShapeDtypeStruct((1, seq_len, emb_dim), jnp.float32),
        out_specs=pl.BlockSpec(memory_space=pl.ANY),
        scratch_shapes=[pltpu.SemaphoreType.DMA],
        compiler_params=pltpu.CompilerParams(
            disable_bounds_checks=True,
            disable_semaphore_checks=True,
            skip_device_barrier=True,
        ),
    )


def kernel(x, wpe):
    seq_len = x.shape[1]
    emb_dim = wpe.shape[1]
    return _tc_copy(seq_len, emb_dim)(wpe[:seq_len])


# speedup vs baseline: 1.0071x; 1.0018x over previous
import functools

import jax
import jax.numpy as jnp
from jax.experimental import pallas as pl
from jax.experimental.pallas import tpu as pltpu


def _copy_body(w_ref, o_ref, sem):
    copy = pltpu.make_async_copy(w_ref, o_ref.at[0], sem)
    copy.start()
    copy.wait()


@functools.cache
def _tc_copy(seq_len: int, emb_dim: int):
    return pl.pallas_call(
        _copy_body,
        out_shape=jax.ShapeDtypeStruct((1, seq_len, emb_dim), jnp.float32),
        in_specs=[pl.BlockSpec(memory_space=pl.ANY)],
        scratch_shapes=[pltpu.SemaphoreType.DMA],
        compiler_params=pltpu.CompilerParams(
            disable_bounds_checks=True,
            disable_semaphore_checks=True,
            skip_device_barrier=True,
        ),
    )


def kernel(x, wpe):
    seq_len = x.shape[1]
    emb_dim = wpe.shape[1]
    return _tc_copy(seq_len, emb_dim)(wpe[:seq_len])
